# baseline (speedup 1.0000x reference)
import functools

import jax
import jax.numpy as jnp
from jax.experimental import pallas as pl
from jax.experimental.pallas import tpu as pltpu

NUM_LABELED = 18048
OUT_CH = 4096
INV_TEMP = 20.0
ROW_BLOCK = 1128
GRID = NUM_LABELED // ROW_BLOCK


def _fused_kernel(pids_ref, feat_cat_ref, features1_ref, features_ref,
                  mem_ref, out_ref, acc_sumexp, acc_picked, acc_small):
    g = pl.program_id(0)

    @pl.when(g == 0)
    def _init():
        acc_sumexp[:, :] = jnp.zeros_like(acc_sumexp)
        acc_picked[:, :] = jnp.zeros_like(acc_picked)
        f = features_ref[:, :]
        f1 = features1_ref[:, :]
        loss_cos = 1.0 - jnp.mean(jnp.sum(f * f1, axis=1))
        sim = jax.lax.dot_general(f, f, (((1,), (1,)), ((), ())),
                                  preferred_element_type=jnp.float32)
        sim1 = jax.lax.dot_general(f1, f1, (((1,), (1,)), ((), ())),
                                   preferred_element_type=jnp.float32)
        lse_s = jnp.log(jnp.sum(jnp.exp(sim), axis=1, keepdims=True))
        lse_s1 = jnp.log(jnp.sum(jnp.exp(sim1), axis=1, keepdims=True))
        log_p = sim - lse_s
        log_q = sim1 - lse_s1
        p = jnp.exp(log_p)
        q = jnp.exp(log_q)
        loss_kl = jnp.sum(q * (log_q - log_p))
        loss_kl1 = jnp.sum(p * (log_p - log_q))
        acc_small[:, :] = (loss_cos + loss_kl + loss_kl1).reshape(1, 1)

    logits = jax.lax.dot_general(
        feat_cat_ref[:, :], mem_ref[:, :],
        (((1,), (1,)), ((), ())),
        preferred_element_type=jnp.float32) * INV_TEMP

    acc_sumexp[:, :] += jnp.sum(jnp.exp(logits), axis=1, keepdims=True)

    col = g * ROW_BLOCK + jax.lax.broadcasted_iota(jnp.int32, (32, ROW_BLOCK), 1)
    pids = pids_ref[:, :]
    acc_picked[:, :] += jnp.sum(jnp.where(col == pids, logits, 0.0),
                                axis=1, keepdims=True)

    @pl.when(g == GRID - 1)
    def _finalize():
        lse = jnp.log(acc_sumexp[:, :])
        loss = jnp.mean(lse - acc_picked[:, :])
        loss1 = jnp.log(jnp.float32(NUM_LABELED))
        out_ref[:, :] = (loss + loss1) * 0.5 + acc_small[:, :]


@functools.partial(jax.jit, static_argnames=("interpret",))
def _run(feat_cat, features1, features, pids, memory_features,
         interpret=False):
    total = pl.pallas_call(
        _fused_kernel,
        grid=(GRID,),
        in_specs=[
            pl.BlockSpec((32, 1), lambda g: (0, 0)),
            pl.BlockSpec((32, OUT_CH), lambda g: (0, 0)),
            pl.BlockSpec((32, OUT_CH), lambda g: (0, 0)),
            pl.BlockSpec((32, OUT_CH), lambda g: (0, 0)),
            pl.BlockSpec((ROW_BLOCK, OUT_CH), lambda g: (g, 0)),
        ],
        out_specs=pl.BlockSpec((1, 1), lambda g: (0, 0)),
        out_shape=jax.ShapeDtypeStruct((1, 1), jnp.float32),
        scratch_shapes=[
            pltpu.VMEM((32, 1), jnp.float32),
            pltpu.VMEM((32, 1), jnp.float32),
            pltpu.VMEM((1, 1), jnp.float32),
        ],
        compiler_params=pltpu.CompilerParams(
            dimension_semantics=("arbitrary",)),
        interpret=interpret,
    )(pids, feat_cat, features1, features, memory_features)
    return total[0, 0]


def kernel(feat_cat, features1, features, gt_labels, memory_features):
    pids = gt_labels[..., -2].reshape(32, 1).astype(jnp.int32)
    return _run(feat_cat, features1, features, pids, memory_features)


# speedup vs baseline: 1.0124x; 1.0124x over previous
import functools

import jax
import jax.numpy as jnp
from jax.experimental import pallas as pl
from jax.experimental.pallas import tpu as pltpu

NUM_LABELED = 18048
OUT_CH = 4096
INV_TEMP = 20.0
ROW_BLOCK = 752
GRID = NUM_LABELED // ROW_BLOCK


def _fused_kernel(pids_ref, feat_cat_ref, features1_ref, features_ref,
                  mem_ref, out_ref, acc_sumexp, acc_picked, acc_small):
    g = pl.program_id(0)

    @pl.when(g == 0)
    def _init():
        acc_sumexp[:, :] = jnp.zeros_like(acc_sumexp)
        acc_picked[:, :] = jnp.zeros_like(acc_picked)
        f = features_ref[:, :]
        f1 = features1_ref[:, :]
        loss_cos = 1.0 - jnp.mean(jnp.sum(f * f1, axis=1))
        sim = jax.lax.dot_general(f, f, (((1,), (1,)), ((), ())),
                                  preferred_element_type=jnp.float32)
        sim1 = jax.lax.dot_general(f1, f1, (((1,), (1,)), ((), ())),
                                   preferred_element_type=jnp.float32)
        lse_s = jnp.log(jnp.sum(jnp.exp(sim), axis=1, keepdims=True))
        lse_s1 = jnp.log(jnp.sum(jnp.exp(sim1), axis=1, keepdims=True))
        log_p = sim - lse_s
        log_q = sim1 - lse_s1
        p = jnp.exp(log_p)
        q = jnp.exp(log_q)
        loss_kl = jnp.sum(q * (log_q - log_p))
        loss_kl1 = jnp.sum(p * (log_p - log_q))
        acc_small[:, :] = (loss_cos + loss_kl + loss_kl1).reshape(1, 1)

    logits = jax.lax.dot_general(
        feat_cat_ref[:, :], mem_ref[:, :],
        (((1,), (1,)), ((), ())),
        preferred_element_type=jnp.float32) * INV_TEMP

    acc_sumexp[:, :] += jnp.sum(jnp.exp(logits), axis=1, keepdims=True)

    col = g * ROW_BLOCK + jax.lax.broadcasted_iota(jnp.int32, (32, ROW_BLOCK), 1)
    pids = pids_ref[:, :]
    acc_picked[:, :] += jnp.sum(jnp.where(col == pids, logits, 0.0),
                                axis=1, keepdims=True)

    @pl.when(g == GRID - 1)
    def _finalize():
        lse = jnp.log(acc_sumexp[:, :])
        loss = jnp.mean(lse - acc_picked[:, :])
        loss1 = jnp.log(jnp.float32(NUM_LABELED))
        out_ref[:, :] = (loss + loss1) * 0.5 + acc_small[:, :]


@functools.partial(jax.jit, static_argnames=("interpret",))
def _run(feat_cat, features1, features, pids, memory_features,
         interpret=False):
    total = pl.pallas_call(
        _fused_kernel,
        grid=(GRID,),
        in_specs=[
            pl.BlockSpec((32, 1), lambda g: (0, 0)),
            pl.BlockSpec((32, OUT_CH), lambda g: (0, 0)),
            pl.BlockSpec((32, OUT_CH), lambda g: (0, 0)),
            pl.BlockSpec((32, OUT_CH), lambda g: (0, 0)),
            pl.BlockSpec((ROW_BLOCK, OUT_CH), lambda g: (g, 0)),
        ],
        out_specs=pl.BlockSpec((1, 1), lambda g: (0, 0)),
        out_shape=jax.ShapeDtypeStruct((1, 1), jnp.float32),
        scratch_shapes=[
            pltpu.VMEM((32, 1), jnp.float32),
            pltpu.VMEM((32, 1), jnp.float32),
            pltpu.VMEM((1, 1), jnp.float32),
        ],
        compiler_params=pltpu.CompilerParams(
            dimension_semantics=("arbitrary",)),
        interpret=interpret,
    )(pids, feat_cat, features1, features, memory_features)
    return total[0, 0]


def kernel(feat_cat, features1, features, gt_labels, memory_features):
    pids = gt_labels[..., -2].reshape(32, 1).astype(jnp.int32)
    return _run(feat_cat, features1, features, pids, memory_features)
